# Initial kernel scaffold; baseline (speedup 1.0000x reference)
import jax
import jax.numpy as jnp
from jax.experimental import pallas as pl


def kernel(node_attr, edgeij_pair, edge_attr, g, batch):
    raise NotImplementedError("write your pallas kernel here")


# speedup vs baseline: 5.1230x; 5.1230x over previous
import functools

import jax
import jax.numpy as jnp
from jax import lax
from jax.experimental import pallas as pl
from jax.experimental.pallas import tpu as pltpu
from jax.experimental.pallas import tpu_sc as plsc

_N_NODES = 10000
_N_EDGES = 320000
_D_FEAT = 128

_NC = 2
_NS = 16
_NW = _NC * _NS
_LANE = 128
_CHUNKS = 79
_E_PAD = _NW * _CHUNKS * _LANE
_N_PAD = 10240
_ZPT = _N_PAD // _NS

_sc_mesh = plsc.VectorSubcoreMesh(
    core_axis_name="c", subcore_axis_name="s", num_cores=_NC, num_subcores=_NS
)


@functools.partial(
    pl.kernel,
    out_type=jax.ShapeDtypeStruct((_NC, _N_PAD), jnp.float32),
    mesh=_sc_mesh,
    scratch_types=[
        pltpu.VMEM((_CHUNKS, _LANE), jnp.int32),
        pltpu.VMEM((_CHUNKS, _LANE), jnp.float32),
        pltpu.VMEM((_ZPT,), jnp.float32),
        pltpu.VMEM_SHARED((_N_PAD,), jnp.float32),
    ],
)
def _sc_segment_sum(dst_hbm, val_hbm, out_hbm, idx_v, val_v, zero_v, acc_sh):
    c = lax.axis_index("c")
    s = lax.axis_index("s")
    wid = s * _NC + c

    pltpu.sync_copy(dst_hbm.at[wid], idx_v)
    pltpu.sync_copy(val_hbm.at[wid], val_v)

    for i in range(_ZPT // 16):
        zero_v[pl.ds(i * 16, 16)] = jnp.zeros((16,), jnp.float32)
    pltpu.sync_copy(zero_v, acc_sh.at[pl.ds(s * _ZPT, _ZPT)])
    plsc.subcore_barrier()

    def body(j, carry):
        pltpu.sync_copy(val_v.at[j], acc_sh.at[idx_v.at[j]], add=True)
        return carry

    lax.fori_loop(0, _CHUNKS, body, 0)
    plsc.subcore_barrier()

    @pl.when(s == 0)
    def _():
        pltpu.sync_copy(acc_sh, out_hbm.at[c])


_BLK = 1000


def _tc_body(x_ref, p0_ref, p1_ref, o_ref):
    x = x_ref[...]
    cbar = p0_ref[...] + p1_ref[...]
    o_ref[:, :_D_FEAT] = x
    o_ref[:, _D_FEAT:] = x * cbar


def _tc_fuse(x, p0, p1):
    grid = (_N_NODES // _BLK,)
    return pl.pallas_call(
        _tc_body,
        grid=grid,
        in_specs=[
            pl.BlockSpec((_BLK, _D_FEAT), lambda i: (i, 0)),
            pl.BlockSpec((_BLK, 1), lambda i: (i, 0)),
            pl.BlockSpec((_BLK, 1), lambda i: (i, 0)),
        ],
        out_specs=pl.BlockSpec((_BLK, 2 * _D_FEAT), lambda i: (i, 0)),
        out_shape=jax.ShapeDtypeStruct((_N_NODES, 2 * _D_FEAT), jnp.float32),
    )(x, p0, p1)


def kernel(node_attr, edgeij_pair, edge_attr, g, batch):
    dst = edgeij_pair[0]
    vals = edge_attr[:, 1]
    pad = _E_PAD - _N_EDGES
    dst_p = jnp.pad(dst, (0, pad)).reshape(_NW, _CHUNKS, _LANE)
    vals_p = jnp.pad(vals, (0, pad)).reshape(_NW, _CHUNKS, _LANE)
    partials = _sc_segment_sum(dst_p, vals_p)
    p0 = partials[0, :_N_NODES].reshape(_N_NODES, 1)
    p1 = partials[1, :_N_NODES].reshape(_N_NODES, 1)
    return _tc_fuse(node_attr, p0, p1)
